# Initial kernel scaffold; baseline (speedup 1.0000x reference)
import jax
import jax.numpy as jnp
from jax.experimental import pallas as pl


def kernel(x, edge_index, ring_polarities, restriction_w, send_maps, receive_maps, delta_send, delta_receive, w1, b1, ln1_w, ln1_b, w2, b2, norm_w, norm_b, res_scale):
    raise NotImplementedError("write your pallas kernel here")


# speedup vs baseline: 12.7665x; 12.7665x over previous
import functools
import jax
import jax.numpy as jnp
from jax import lax
from jax.experimental import pallas as pl
from jax.experimental.pallas import tpu as pltpu
from jax.experimental.pallas import tpu_sc as plsc

N = 50000
E = 400000
D = 32
P = 9

NC = 2
NS = 16
NW = NC * NS
CH = 128
NCHUNK = 98
EPW = NCHUNK * CH
EP = NW * EPW
NPAD = 50048
RPT = NPAD // NS

NB = 2000
GRID = N // NB

_f32 = jnp.float32


def _send_body(x_ref, pol_ref, sm_ref, ds_ref, rw_ref, o_ref):
    xb = x_ref[...]
    pol = lax.rem(pol_ref[...], P)
    rw = rw_ref[...]
    a_all = jnp.concatenate(
        [lax.dot_general(sm_ref[p] + ds_ref[p], rw,
                         (((1,), (1,)), ((), ())),
                         preferred_element_type=_f32)
         for p in range(P)], axis=1)
    y = jnp.dot(xb, a_all, preferred_element_type=_f32)
    acc = jnp.zeros((NB, D), _f32)
    for p in range(P):
        m = (pol == p).astype(_f32)
        acc = acc + m * y[:, p * D:(p + 1) * D]
    o_ref[...] = acc


def _send_transform(x, pol2, send_maps, delta_send, restriction_w):
    full = lambda shape: pl.BlockSpec(shape, lambda i: (0,) * len(shape))
    return pl.pallas_call(
        _send_body,
        grid=(GRID,),
        in_specs=[
            pl.BlockSpec((NB, D), lambda i: (i, 0)),
            pl.BlockSpec((NB, 1), lambda i: (i, 0)),
            full((P, D, D)), full((P, D, D)), full((D, D)),
        ],
        out_specs=pl.BlockSpec((NB, D), lambda i: (i, 0)),
        out_shape=jax.ShapeDtypeStruct((N, D), _f32),
    )(x, pol2, send_maps, delta_send, restriction_w)


def _sc_agg_body(xs_hbm, row_hbm, col_hbm, z32_hbm, agg_out,
                 idxr, idxc, rows, sh_agg, sem):
    c = lax.axis_index("c")
    s = lax.axis_index("s")
    wid = s * NC + c

    pltpu.sync_copy(row_hbm.at[wid], idxr)
    pltpu.sync_copy(col_hbm.at[wid], idxc)
    pltpu.sync_copy(z32_hbm, sh_agg.at[pl.ds(s * RPT, RPT)])
    plsc.subcore_barrier()

    def step(j, carry):
        pltpu.async_copy(xs_hbm.at[idxr.at[j]], rows, sem).wait()
        pltpu.sync_copy(rows, sh_agg.at[idxc.at[j]], add=True)
        return carry

    lax.fori_loop(0, NCHUNK, step, 0)
    plsc.subcore_barrier()
    sl = pl.ds(s * RPT, RPT)
    pltpu.sync_copy(sh_agg.at[sl], agg_out.at[c, sl])


@functools.partial(
    pl.kernel,
    mesh=plsc.VectorSubcoreMesh(core_axis_name="c", subcore_axis_name="s"),
    compiler_params=pltpu.CompilerParams(use_tc_tiling_on_sc=False),
    out_type=jax.ShapeDtypeStruct((NC, NPAD, D), _f32),
    scratch_types=[
        pltpu.VMEM((NCHUNK, CH), jnp.int32),
        pltpu.VMEM((NCHUNK, CH), jnp.int32),
        pltpu.VMEM((CH, D), _f32),
        pltpu.VMEM_SHARED((NPAD, D), _f32),
        pltpu.SemaphoreType.DMA,
    ],
)
def _sc_gather_scatter(*refs):
    _sc_agg_body(*refs)


def _sc_deg_body(row_hbm, ones_hbm, z8_hbm, deg_out,
                 idxr, ones_v, sh_deg):
    c = lax.axis_index("c")
    s = lax.axis_index("s")
    wid = s * NC + c

    pltpu.sync_copy(row_hbm.at[wid], idxr)
    pltpu.sync_copy(ones_hbm, ones_v)
    pltpu.sync_copy(z8_hbm, sh_deg.at[pl.ds(s * RPT, RPT)])
    plsc.subcore_barrier()

    def step(j, carry):
        pltpu.sync_copy(ones_v, sh_deg.at[idxr.at[j]], add=True)
        return carry

    lax.fori_loop(0, NCHUNK, step, 0)
    plsc.subcore_barrier()
    sl = pl.ds(s * RPT, RPT)
    pltpu.sync_copy(sh_deg.at[sl], deg_out.at[c, sl])


@functools.partial(
    pl.kernel,
    mesh=plsc.VectorSubcoreMesh(core_axis_name="c", subcore_axis_name="s"),
    compiler_params=pltpu.CompilerParams(use_tc_tiling_on_sc=False),
    out_type=jax.ShapeDtypeStruct((NC, NPAD, 8), _f32),
    scratch_types=[
        pltpu.VMEM((NCHUNK, CH), jnp.int32),
        pltpu.VMEM((CH, 8), _f32),
        pltpu.VMEM_SHARED((NPAD, 8), _f32),
    ],
)
def _sc_degree(*refs):
    _sc_deg_body(*refs)


def _recv_body(a0_ref, a1_ref, d0_ref, d1_ref, x_ref, pol_ref,
               rm_ref, dr_ref, w1_ref, b1_ref, ln1w_ref, ln1b_ref,
               w2_ref, b2_ref, nw_ref, nb_ref, res_ref, o_ref):
    agg = a0_ref[...] + a1_ref[...]
    deg = (d0_ref[...] + d1_ref[...])[:, 0:1]
    pol = lax.rem(pol_ref[...], P)
    r_all = jnp.concatenate(
        [rm_ref[p] + dr_ref[p] for p in range(P)], axis=1)
    y = jnp.dot(agg, r_all, preferred_element_type=_f32)
    out = jnp.zeros((NB, D), _f32)
    for p in range(P):
        m = (pol == p).astype(_f32)
        out = out + m * y[:, p * D:(p + 1) * D]
    out = out / jnp.maximum(deg, 1.0)

    def layernorm(h, w, b):
        mu = jnp.mean(h, axis=1, keepdims=True)
        var = jnp.mean((h - mu) ** 2, axis=1, keepdims=True)
        return (h - mu) * lax.rsqrt(var + 1e-5) * w + b

    h = lax.dot_general(out, w1_ref[...], (((1,), (1,)), ((), ())),
                        preferred_element_type=_f32) + b1_ref[...]
    h = layernorm(h, ln1w_ref[...], ln1b_ref[...])
    h = jnp.maximum(h, 0.0)
    h = lax.dot_general(h, w2_ref[...], (((1,), (1,)), ((), ())),
                        preferred_element_type=_f32) + b2_ref[...]
    h = res_ref[0, 0] * h + x_ref[...]
    o_ref[...] = layernorm(h, nw_ref[...], nb_ref[...])


def _recv_update(a0, a1, d0, d1, x, pol2, receive_maps, delta_receive,
                 w1, b1, ln1_w, ln1_b, w2, b2, norm_w, norm_b, res_scale):
    full = lambda shape: pl.BlockSpec(shape, lambda i: (0,) * len(shape))
    return pl.pallas_call(
        _recv_body,
        grid=(GRID,),
        in_specs=[
            pl.BlockSpec((NB, D), lambda i: (i, 0)),
            pl.BlockSpec((NB, D), lambda i: (i, 0)),
            pl.BlockSpec((NB, 8), lambda i: (i, 0)),
            pl.BlockSpec((NB, 8), lambda i: (i, 0)),
            pl.BlockSpec((NB, D), lambda i: (i, 0)),
            pl.BlockSpec((NB, 1), lambda i: (i, 0)),
            full((P, D, D)), full((P, D, D)),
            full((D, D)), full((1, D)), full((1, D)), full((1, D)),
            full((D, D)), full((1, D)), full((1, D)), full((1, D)),
            full((1, 1)),
        ],
        out_specs=pl.BlockSpec((NB, D), lambda i: (i, 0)),
        out_shape=jax.ShapeDtypeStruct((N, D), _f32),
    )(a0, a1, d0, d1, x, pol2, receive_maps, delta_receive,
      w1, b1, ln1_w, ln1_b, w2, b2, norm_w, norm_b, res_scale)


def kernel(x, edge_index, ring_polarities, restriction_w, send_maps,
           receive_maps, delta_send, delta_receive, w1, b1, ln1_w, ln1_b,
           w2, b2, norm_w, norm_b, res_scale):
    pol2 = ring_polarities.reshape(N, 1)
    xs = _send_transform(x, pol2, send_maps, delta_send, restriction_w)
    xs_pad = jnp.concatenate([xs, jnp.zeros((NPAD - N, D), _f32)], axis=0)

    padv = jnp.full((EP - E,), N, jnp.int32)
    row3 = jnp.concatenate([edge_index[0], padv]).reshape(NW, NCHUNK, CH)
    col3 = jnp.concatenate([edge_index[1], padv]).reshape(NW, NCHUNK, CH)
    ones_h = jnp.ones((CH, 8), _f32)
    z32 = jnp.zeros((RPT, D), _f32)
    z8 = jnp.zeros((RPT, 8), _f32)

    agg2 = _sc_gather_scatter(xs_pad, row3, col3, z32)
    deg2 = _sc_degree(row3, ones_h, z8)

    return _recv_update(
        agg2[0, :N], agg2[1, :N], deg2[0, :N], deg2[1, :N], x, pol2,
        receive_maps, delta_receive,
        w1, b1.reshape(1, D), ln1_w.reshape(1, D), ln1_b.reshape(1, D),
        w2, b2.reshape(1, D), norm_w.reshape(1, D), norm_b.reshape(1, D),
        jnp.asarray(res_scale, _f32).reshape(1, 1))


# speedup vs baseline: 14.8579x; 1.1638x over previous
import functools
import jax
import jax.numpy as jnp
from jax import lax
from jax.experimental import pallas as pl
from jax.experimental.pallas import tpu as pltpu
from jax.experimental.pallas import tpu_sc as plsc

N = 50000
E = 400000
D = 32
P = 9
PD = P * D

NC = 2
NS = 16
NW = NC * NS
CH = 64
NCHUNK = 196
EPW = NCHUNK * CH
EP = NW * EPW
NPAD = 50048
RPT = NPAD // NS

NBA = 3128
GRIDA = NPAD // NBA
NB = 2000
GRID = N // NB

_f32 = jnp.float32


def _pol_mask(pol):
    pat = lax.broadcasted_iota(jnp.int32, (1, PD), 1) // D
    return (lax.rem(pol, P) == pat).astype(_f32)


def _send_body(x_ref, pol_ref, sm_ref, ds_ref, rw_ref, o_ref):
    xb = x_ref[...]
    rw = rw_ref[...]
    a_all = jnp.concatenate(
        [lax.dot_general(sm_ref[p] + ds_ref[p], rw,
                         (((1,), (1,)), ((), ())),
                         preferred_element_type=_f32)
         for p in range(P)], axis=1)
    y = jnp.dot(xb, a_all, preferred_element_type=_f32)
    y = y * _pol_mask(pol_ref[...])
    acc = jnp.zeros((NBA, D), _f32)
    for p in range(P):
        acc = acc + y[:, p * D:(p + 1) * D]
    o_ref[...] = acc


def _send_transform(x_p, pol_p, send_maps, delta_send, restriction_w):
    full = lambda shape: pl.BlockSpec(shape, lambda i: (0,) * len(shape))
    return pl.pallas_call(
        _send_body,
        grid=(GRIDA,),
        in_specs=[
            pl.BlockSpec((NBA, D), lambda i: (i, 0)),
            pl.BlockSpec((NBA, 1), lambda i: (i, 0)),
            full((P, D, D)), full((P, D, D)), full((D, D)),
        ],
        out_specs=pl.BlockSpec((NBA, D), lambda i: (i, 0)),
        out_shape=jax.ShapeDtypeStruct((NPAD, D), _f32),
    )(x_p, pol_p, send_maps, delta_send, restriction_w)


def _sc_agg_body(xs_hbm, row_hbm, col_hbm, z32_hbm, agg_out,
                 idxr, idxc, rows, sh_agg, gs0, gs1):
    c = lax.axis_index("c")
    s = lax.axis_index("s")
    wid = s * NC + c

    pltpu.sync_copy(row_hbm.at[wid], idxr)
    pltpu.sync_copy(col_hbm.at[wid], idxc)
    pltpu.sync_copy(z32_hbm, sh_agg.at[pl.ds(s * RPT, RPT)])
    plsc.subcore_barrier()

    def gather(j, b, sem):
        pltpu.async_copy(xs_hbm.at[idxr.at[j]], rows.at[b], sem)

    def wait_gather(b, sem):
        pltpu.make_async_copy(xs_hbm.at[idxr.at[0]], rows.at[b], sem).wait()

    def scatter(j, b):
        pltpu.sync_copy(rows.at[b], sh_agg.at[idxc.at[j]], add=True)

    gather(0, 0, gs0)

    def pair(i, carry):
        j0 = i * 2
        wait_gather(0, gs0)
        gather(j0 + 1, 1, gs1)
        scatter(j0, 0)

        wait_gather(1, gs1)

        @pl.when(j0 + 2 < NCHUNK)
        def _():
            gather(j0 + 2, 0, gs0)
        scatter(j0 + 1, 1)
        return carry

    lax.fori_loop(0, NCHUNK // 2, pair, 0)

    plsc.subcore_barrier()
    sl = pl.ds(s * RPT, RPT)
    pltpu.sync_copy(sh_agg.at[sl], agg_out.at[c, sl])


@functools.partial(
    pl.kernel,
    mesh=plsc.VectorSubcoreMesh(core_axis_name="c", subcore_axis_name="s"),
    compiler_params=pltpu.CompilerParams(use_tc_tiling_on_sc=False),
    out_type=jax.ShapeDtypeStruct((NC, NPAD, D), _f32),
    scratch_types=[
        pltpu.VMEM((NCHUNK, CH), jnp.int32),
        pltpu.VMEM((NCHUNK, CH), jnp.int32),
        pltpu.VMEM((2, CH, D), _f32),
        pltpu.VMEM_SHARED((NPAD, D), _f32),
        pltpu.SemaphoreType.DMA,
        pltpu.SemaphoreType.DMA,
    ],
)
def _sc_gather_scatter(*refs):
    _sc_agg_body(*refs)


def _sc_deg_body(row_hbm, ones_hbm, z8_hbm, deg_out, idxr, ones_v, sh_deg):
    c = lax.axis_index("c")
    s = lax.axis_index("s")
    wid = s * NC + c

    pltpu.sync_copy(row_hbm.at[wid], idxr)
    pltpu.sync_copy(ones_hbm, ones_v)
    pltpu.sync_copy(z8_hbm, sh_deg.at[pl.ds(s * RPT, RPT)])
    plsc.subcore_barrier()

    def step(j, carry):
        pltpu.sync_copy(ones_v, sh_deg.at[idxr.at[j]], add=True)
        return carry

    lax.fori_loop(0, NCHUNK, step, 0)
    plsc.subcore_barrier()
    sl = pl.ds(s * RPT, RPT)
    pltpu.sync_copy(sh_deg.at[sl], deg_out.at[c, sl])


@functools.partial(
    pl.kernel,
    mesh=plsc.VectorSubcoreMesh(core_axis_name="c", subcore_axis_name="s"),
    compiler_params=pltpu.CompilerParams(use_tc_tiling_on_sc=False),
    out_type=jax.ShapeDtypeStruct((NC, NPAD, 8), _f32),
    scratch_types=[
        pltpu.VMEM((NCHUNK, CH), jnp.int32),
        pltpu.VMEM((CH, 8), _f32),
        pltpu.VMEM_SHARED((NPAD, 8), _f32),
    ],
)
def _sc_degree(*refs):
    _sc_deg_body(*refs)


def _recv_body(a_ref, d_ref, x_ref, pol_ref,
               rm_ref, dr_ref, w1_ref, b1_ref, ln1w_ref, ln1b_ref,
               w2_ref, b2_ref, nw_ref, nb_ref, res_ref, o_ref):
    agg = a_ref[0] + a_ref[1]
    deg = (d_ref[0] + d_ref[1])[:, 0:1]
    r_all = jnp.concatenate(
        [rm_ref[p] + dr_ref[p] for p in range(P)], axis=1)
    y = jnp.dot(agg, r_all, preferred_element_type=_f32)
    y = y * _pol_mask(pol_ref[...])
    out = jnp.zeros((NB, D), _f32)
    for p in range(P):
        out = out + y[:, p * D:(p + 1) * D]
    out = out / jnp.maximum(deg, 1.0)

    def layernorm(h, w, b):
        mu = jnp.mean(h, axis=1, keepdims=True)
        var = jnp.mean((h - mu) ** 2, axis=1, keepdims=True)
        return (h - mu) * lax.rsqrt(var + 1e-5) * w + b

    h = lax.dot_general(out, w1_ref[...], (((1,), (1,)), ((), ())),
                        preferred_element_type=_f32) + b1_ref[...]
    h = layernorm(h, ln1w_ref[...], ln1b_ref[...])
    h = jnp.maximum(h, 0.0)
    h = lax.dot_general(h, w2_ref[...], (((1,), (1,)), ((), ())),
                        preferred_element_type=_f32) + b2_ref[...]
    h = res_ref[0, 0] * h + x_ref[...]
    o_ref[...] = layernorm(h, nw_ref[...], nb_ref[...])


def _recv_update(agg2, deg2, x, pol2, receive_maps, delta_receive,
                 w1, b1, ln1_w, ln1_b, w2, b2, norm_w, norm_b, res_scale):
    full = lambda shape: pl.BlockSpec(shape, lambda i: (0,) * len(shape))
    return pl.pallas_call(
        _recv_body,
        grid=(GRID,),
        in_specs=[
            pl.BlockSpec((NC, NB, D), lambda i: (0, i, 0)),
            pl.BlockSpec((NC, NB, 8), lambda i: (0, i, 0)),
            pl.BlockSpec((NB, D), lambda i: (i, 0)),
            pl.BlockSpec((NB, 1), lambda i: (i, 0)),
            full((P, D, D)), full((P, D, D)),
            full((D, D)), full((1, D)), full((1, D)), full((1, D)),
            full((D, D)), full((1, D)), full((1, D)), full((1, D)),
            full((1, 1)),
        ],
        out_specs=pl.BlockSpec((NB, D), lambda i: (i, 0)),
        out_shape=jax.ShapeDtypeStruct((N, D), _f32),
    )(agg2, deg2, x, pol2, receive_maps, delta_receive,
      w1, b1, ln1_w, ln1_b, w2, b2, norm_w, norm_b, res_scale)


def kernel(x, edge_index, ring_polarities, restriction_w, send_maps,
           receive_maps, delta_send, delta_receive, w1, b1, ln1_w, ln1_b,
           w2, b2, norm_w, norm_b, res_scale):
    pol2 = ring_polarities.reshape(N, 1)
    x_p = jnp.pad(x, ((0, NPAD - N), (0, 0)))
    pol_p = jnp.pad(pol2, ((0, NPAD - N), (0, 0)))

    xs = _send_transform(x_p, pol_p, send_maps, delta_send, restriction_w)

    edges4 = jnp.pad(edge_index, ((0, 0), (0, EP - E)),
                     constant_values=N).reshape(2, NW, NCHUNK, CH)
    ones_h = jnp.ones((CH, 8), _f32)
    z32 = jnp.zeros((RPT, D), _f32)
    z8 = jnp.zeros((RPT, 8), _f32)

    agg2 = _sc_gather_scatter(xs, edges4[0], edges4[1], z32)
    deg2 = _sc_degree(edges4[0], ones_h, z8)

    return _recv_update(
        agg2, deg2, x, pol2, receive_maps, delta_receive,
        w1, b1.reshape(1, D), ln1_w.reshape(1, D), ln1_b.reshape(1, D),
        w2, b2.reshape(1, D), norm_w.reshape(1, D), norm_b.reshape(1, D),
        jnp.asarray(res_scale, _f32).reshape(1, 1))


# speedup vs baseline: 17.3092x; 1.1650x over previous
import functools
import jax
import jax.numpy as jnp
from jax import lax
from jax.experimental import pallas as pl
from jax.experimental.pallas import tpu as pltpu
from jax.experimental.pallas import tpu_sc as plsc

N = 50000
E = 400000
D = 32
P = 9
PD = P * D

NC = 2
NS = 16
NW = NC * NS
CH = 128
NCHUNK = 98
GC = 14
NG = NCHUNK // GC
EPW = NCHUNK * CH
EP = NW * EPW
NPAD = 50048
RPT = NPAD // NS

NBA = 2000
GRIDA = N // NBA
NB = 2000
GRID = N // NB

_f32 = jnp.float32


def _pol_mask(pol):
    pat = lax.broadcasted_iota(jnp.int32, (1, PD), 1) // D
    return (lax.rem(pol, P) == pat).astype(_f32)


def _send_body(x_ref, pol_ref, sm_ref, ds_ref, rw_ref, o_ref):
    xb = x_ref[...]
    rw = rw_ref[...]
    a_all = jnp.concatenate(
        [lax.dot_general(sm_ref[p] + ds_ref[p], rw,
                         (((1,), (1,)), ((), ())),
                         preferred_element_type=_f32)
         for p in range(P)], axis=1)
    y = jnp.dot(xb, a_all, preferred_element_type=_f32)
    y = y * _pol_mask(pol_ref[...])
    acc = jnp.zeros((NBA, D), _f32)
    for p in range(P):
        acc = acc + y[:, p * D:(p + 1) * D]
    o_ref[...] = acc


def _send_transform(x, pol2, send_maps, delta_send, restriction_w):
    full = lambda shape: pl.BlockSpec(shape, lambda i: (0,) * len(shape))
    return pl.pallas_call(
        _send_body,
        grid=(GRIDA,),
        in_specs=[
            pl.BlockSpec((NBA, D), lambda i: (i, 0)),
            pl.BlockSpec((NBA, 1), lambda i: (i, 0)),
            full((P, D, D)), full((P, D, D)), full((D, D)),
        ],
        out_specs=pl.BlockSpec((NBA, D), lambda i: (i, 0)),
        out_shape=jax.ShapeDtypeStruct((NPAD, D), _f32),
    )(x, pol2, send_maps, delta_send, restriction_w)


def _sc_agg_body(xs_hbm, row_hbm, col_hbm, z32_hbm, agg_out,
                 idxr, idxc, rows, sh_agg, gs0, gs1, ss0, ss1, cs):
    c = lax.axis_index("c")
    s = lax.axis_index("s")
    wid = s * NC + c

    pltpu.sync_copy(row_hbm.at[pl.ds(wid * EPW, EPW)], idxr)
    pltpu.async_copy(col_hbm.at[pl.ds(wid * NCHUNK, GC)], idxc.at[0], cs)
    pltpu.sync_copy(z32_hbm, sh_agg.at[pl.ds(s * RPT, RPT)])
    plsc.subcore_barrier()

    def gather(j, b, sem):
        pltpu.async_copy(xs_hbm.at[idxr.at[pl.ds(j * CH, CH)]],
                         rows.at[b], sem)

    def wait_gather(b, sem):
        pltpu.make_async_copy(xs_hbm.at[idxr.at[pl.ds(0, CH)]],
                              rows.at[b], sem).wait()

    def scatter(j, b, sem):
        slot = lax.rem(j // GC, 2)
        pltpu.async_copy(rows.at[b],
                         sh_agg.at[idxc.at[slot, lax.rem(j, GC)]],
                         sem, add=True)

    def wait_scatter(b, sem):
        pltpu.make_async_copy(rows.at[b], sh_agg.at[idxc.at[0, 0]],
                              sem).wait()

    def wait_colgroup():
        pltpu.make_async_copy(col_hbm.at[pl.ds(0, GC)], idxc.at[0],
                              cs).wait()

    gather(0, 0, gs0)

    def pair(k, carry):
        j0 = 2 * k

        @pl.when(lax.rem(j0, GC) == 0)
        def _():
            g = j0 // GC
            wait_colgroup()

            @pl.when(g + 1 < NG)
            def _():
                pltpu.async_copy(
                    col_hbm.at[pl.ds(wid * NCHUNK + (g + 1) * GC, GC)],
                    idxc.at[lax.rem(g + 1, 2)], cs)

        wait_gather(0, gs0)
        scatter(j0, 0, ss0)

        @pl.when(k >= 1)
        def _():
            wait_scatter(1, ss1)
        gather(j0 + 1, 1, gs1)

        wait_gather(1, gs1)
        scatter(j0 + 1, 1, ss1)
        wait_scatter(0, ss0)

        @pl.when(j0 + 2 < NCHUNK)
        def _():
            gather(j0 + 2, 0, gs0)
        return carry

    lax.fori_loop(0, NCHUNK // 2, pair, 0)
    wait_scatter(1, ss1)

    plsc.subcore_barrier()
    sl = pl.ds(s * RPT, RPT)
    pltpu.sync_copy(sh_agg.at[sl], agg_out.at[c, sl])


@functools.partial(
    pl.kernel,
    mesh=plsc.VectorSubcoreMesh(core_axis_name="c", subcore_axis_name="s"),
    compiler_params=pltpu.CompilerParams(use_tc_tiling_on_sc=False),
    out_type=jax.ShapeDtypeStruct((NC, NPAD, D), _f32),
    scratch_types=[
        pltpu.VMEM((EPW,), jnp.int32),
        pltpu.VMEM((2, GC, CH), jnp.int32),
        pltpu.VMEM((2, CH, D), _f32),
        pltpu.VMEM_SHARED((NPAD, D), _f32),
        pltpu.SemaphoreType.DMA,
        pltpu.SemaphoreType.DMA,
        pltpu.SemaphoreType.DMA,
        pltpu.SemaphoreType.DMA,
        pltpu.SemaphoreType.DMA,
    ],
)
def _sc_gather_scatter(*refs):
    _sc_agg_body(*refs)


def _sc_deg_body(row_hbm, ones_hbm, z8_hbm, deg_out, idxr, ones_v, sh_deg):
    c = lax.axis_index("c")
    s = lax.axis_index("s")
    wid = s * NC + c

    pltpu.sync_copy(row_hbm.at[pl.ds(wid * NCHUNK, NCHUNK)], idxr)
    pltpu.sync_copy(ones_hbm, ones_v)
    pltpu.sync_copy(z8_hbm, sh_deg.at[pl.ds(s * RPT, RPT)])
    plsc.subcore_barrier()

    def step(j, carry):
        pltpu.sync_copy(ones_v, sh_deg.at[idxr.at[j]], add=True)
        return carry

    lax.fori_loop(0, NCHUNK, step, 0)
    plsc.subcore_barrier()
    sl = pl.ds(s * RPT, RPT)
    pltpu.sync_copy(sh_deg.at[sl], deg_out.at[c, sl])


@functools.partial(
    pl.kernel,
    mesh=plsc.VectorSubcoreMesh(core_axis_name="c", subcore_axis_name="s"),
    compiler_params=pltpu.CompilerParams(use_tc_tiling_on_sc=False),
    out_type=jax.ShapeDtypeStruct((NC, NPAD, 8), _f32),
    scratch_types=[
        pltpu.VMEM((NCHUNK, CH), jnp.int32),
        pltpu.VMEM((CH, 8), _f32),
        pltpu.VMEM_SHARED((NPAD, 8), _f32),
    ],
)
def _sc_degree(*refs):
    _sc_deg_body(*refs)


def _recv_body(a_ref, d_ref, x_ref, pol_ref,
               rm_ref, dr_ref, w1_ref, b1_ref, ln1w_ref, ln1b_ref,
               w2_ref, b2_ref, nw_ref, nb_ref, res_ref, o_ref):
    agg = a_ref[0] + a_ref[1]
    deg = (d_ref[0] + d_ref[1])[:, 0:1]
    r_all = jnp.concatenate(
        [rm_ref[p] + dr_ref[p] for p in range(P)], axis=1)
    y = jnp.dot(agg, r_all, preferred_element_type=_f32)
    y = y * _pol_mask(pol_ref[...])
    out = jnp.zeros((NB, D), _f32)
    for p in range(P):
        out = out + y[:, p * D:(p + 1) * D]
    out = out / jnp.maximum(deg, 1.0)

    def layernorm(h, w, b):
        mu = jnp.mean(h, axis=1, keepdims=True)
        var = jnp.mean((h - mu) ** 2, axis=1, keepdims=True)
        return (h - mu) * lax.rsqrt(var + 1e-5) * w + b

    h = lax.dot_general(out, w1_ref[...], (((1,), (1,)), ((), ())),
                        preferred_element_type=_f32) + b1_ref[...]
    h = layernorm(h, ln1w_ref[...], ln1b_ref[...])
    h = jnp.maximum(h, 0.0)
    h = lax.dot_general(h, w2_ref[...], (((1,), (1,)), ((), ())),
                        preferred_element_type=_f32) + b2_ref[...]
    h = res_ref[0, 0] * h + x_ref[...]
    o_ref[...] = layernorm(h, nw_ref[...], nb_ref[...])


def _recv_update(agg2, deg2, x, pol2, receive_maps, delta_receive,
                 w1, b1, ln1_w, ln1_b, w2, b2, norm_w, norm_b, res_scale):
    full = lambda shape: pl.BlockSpec(shape, lambda i: (0,) * len(shape))
    return pl.pallas_call(
        _recv_body,
        grid=(GRID,),
        in_specs=[
            pl.BlockSpec((NC, NB, D), lambda i: (0, i, 0)),
            pl.BlockSpec((NC, NB, 8), lambda i: (0, i, 0)),
            pl.BlockSpec((NB, D), lambda i: (i, 0)),
            pl.BlockSpec((NB, 1), lambda i: (i, 0)),
            full((P, D, D)), full((P, D, D)),
            full((D, D)), full((1, D)), full((1, D)), full((1, D)),
            full((D, D)), full((1, D)), full((1, D)), full((1, D)),
            full((1, 1)),
        ],
        out_specs=pl.BlockSpec((NB, D), lambda i: (i, 0)),
        out_shape=jax.ShapeDtypeStruct((N, D), _f32),
    )(agg2, deg2, x, pol2, receive_maps, delta_receive,
      w1, b1, ln1_w, ln1_b, w2, b2, norm_w, norm_b, res_scale)


def kernel(x, edge_index, ring_polarities, restriction_w, send_maps,
           receive_maps, delta_send, delta_receive, w1, b1, ln1_w, ln1_b,
           w2, b2, norm_w, norm_b, res_scale):
    pol2 = ring_polarities.reshape(N, 1)

    xs = _send_transform(x, pol2, send_maps, delta_send, restriction_w)

    row1 = jnp.pad(edge_index[0], (0, EP - E), constant_values=N)
    rowS = row1.reshape(NW * NCHUNK, CH)
    col2 = jnp.pad(edge_index[1], (0, EP - E),
                   constant_values=N).reshape(NW * NCHUNK, CH)
    ones_h = jnp.ones((CH, 8), _f32)
    z32 = jnp.zeros((RPT, D), _f32)
    z8 = jnp.zeros((RPT, 8), _f32)

    agg2 = _sc_gather_scatter(xs, row1, col2, z32)
    deg2 = _sc_degree(rowS, ones_h, z8)

    return _recv_update(
        agg2, deg2, x, pol2, receive_maps, delta_receive,
        w1, b1.reshape(1, D), ln1_w.reshape(1, D), ln1_b.reshape(1, D),
        w2, b2.reshape(1, D), norm_w.reshape(1, D), norm_b.reshape(1, D),
        jnp.asarray(res_scale, _f32).reshape(1, 1))
